# baseline (speedup 1.0000x reference)
import functools

import jax
import jax.numpy as jnp
from jax import lax
from jax.experimental import pallas as pl
from jax.experimental.pallas import tpu as pltpu
from jax.experimental.pallas import tpu_sc as plsc

B = 16
S = 2048
H = S // 2
NUM_MASK = 614
NV = S // 16
NVH = H // 16
MP_PAD = 640
VP_PAD = 1440


def _rotl(v, r):
    return (v << jnp.uint32(r)) | (v >> jnp.uint32(32 - r))


def _tf2x32(k1, k2, x0, x1):
    ks2 = k1 ^ k2 ^ jnp.uint32(0x1BD11BDA)
    x0 = x0 + k1
    x1 = x1 + k2
    rot_a = (13, 15, 26, 6)
    rot_b = (17, 29, 16, 24)
    inj = ((k2, ks2, 1), (ks2, k1, 2), (k1, k2, 3), (k2, ks2, 4), (ks2, k1, 5))
    for i, (ka, kb, c) in enumerate(inj):
        for r in rot_a if i % 2 == 0 else rot_b:
            x0 = x0 + x1
            x1 = _rotl(x1, r)
            x1 = x0 ^ x1
        x0 = x0 + ka
        x1 = x1 + kb + jnp.uint32(c)
    return x0, x1


def _iota16_i32():
    return lax.iota(jnp.int32, 16)


def _tail_pass(kref, vref, k, flip):
    @plsc.parallel_loop(0, NVH, unroll=4)
    def _(b):
        sl = pl.ds(b * 16, 16)
        desc = (((b * 16) & k) != 0) != flip
        dm = jnp.full((16,), jnp.where(desc, jnp.uint32(0xFFFFFFFF),
                                       jnp.uint32(0)))
        ka, va = plsc.sort_key_val(kref[sl] ^ dm, vref[sl])
        kref[sl] = ka ^ dm
        vref[sl] = va


def _pair_pass(kref, vref, k, lj, flip):
    jv = 1 << lj

    @plsc.parallel_loop(0, NVH // 2, unroll=4)
    def _(m):
        a = ((m >> lj) << (lj + 1)) | (m & (jv - 1))
        b2 = a + jv
        asc = (((a * 16) & k) == 0) != flip
        sa = pl.ds(a * 16, 16)
        sb = pl.ds(b2 * 16, 16)
        ka = kref[sa]
        kb = kref[sb]
        va = vref[sa]
        vb = vref[sb]
        take_a = (ka < kb) == jnp.full((16,), asc)
        kref[sa] = jnp.where(take_a, ka, kb)
        kref[sb] = jnp.where(take_a, kb, ka)
        vref[sa] = jnp.where(take_a, va, vb)
        vref[sb] = jnp.where(take_a, vb, va)


def _sort_half(kref, vref, flip):
    _tail_pass(kref, vref, 16, flip)
    for k in (32, 64, 128, 256, 512, 1024):
        lj0 = (k // 32).bit_length() - 1

        def sub_body(t, _, k=k, lj0=lj0):
            _pair_pass(kref, vref, k, lj0 - t, flip)
            return 0

        lax.fori_loop(0, lj0 + 1, sub_body, 0, unroll=False)
        _tail_pass(kref, vref, k, flip)


def _merge_finish(kref, vref):
    def sub_body(t, _):
        _pair_pass(kref, vref, 2048, 5 - t, False)
        return 0

    lax.fori_loop(0, 6, sub_body, 0, unroll=False)
    _tail_pass(kref, vref, 2048, False)


def _body(mask_hbm, mp_hbm, vp_hbm,
          kref, k2ref, vref, kpart, vpart, mref, mpref, vpref,
          shk1, shv1, shk2, shv2, shv3, sem1, sem2):
    c = lax.axis_index("c")
    s = lax.axis_index("s")
    h = s // 8
    row = c * 8 + (s & 7)
    base = h * H
    flip = h == 1

    zero = jnp.zeros((16,), jnp.uint32)
    one = jnp.full((16,), 1, jnp.uint32)

    rk1, rk2 = _tf2x32(
        zero, jnp.full((16,), 42, jnp.uint32), zero,
        jnp.full((16,), row).astype(jnp.uint32))
    s1a, s1b = _tf2x32(rk1, rk2, zero, one)
    n1a, n1b = _tf2x32(rk1, rk2, zero, zero)
    s2a, s2b = _tf2x32(n1a, n1b, zero, one)

    @plsc.parallel_loop(0, NVH, unroll=2)
    def _(i):
        iota = _iota16_i32() + base + i * 16
        counts = iota.astype(jnp.uint32)
        b1, b2 = _tf2x32(s1a, s1b, zero, counts)
        c1, c2 = _tf2x32(s2a, s2b, zero, counts)
        sl = pl.ds(i * 16, 16)
        vref[sl] = iota
        kref[sl] = b1 ^ b2
        k2ref[sl] = c1 ^ c2

    want_min = jnp.full((16,), h == 0)
    partner = s ^ 8

    for rnd in range(2):
        shk, shv = (shk1, shv1) if rnd == 0 else (shk2, shv2)
        if rnd == 1:
            @plsc.parallel_loop(0, NVH, unroll=4)
            def _(i):
                sl = pl.ds(i * 16, 16)
                kref[sl] = k2ref[sl]

        _sort_half(kref, vref, flip)

        cp1 = pltpu.make_async_copy(kref, shk.at[s], sem1)
        cp1.start()
        cp2 = pltpu.make_async_copy(vref, shv.at[s], sem2)
        cp2.start()
        cp1.wait()
        cp2.wait()
        plsc.subcore_barrier()
        cp3 = pltpu.make_async_copy(shk.at[partner], kpart, sem1)
        cp3.start()
        cp4 = pltpu.make_async_copy(shv.at[partner], vpart, sem2)
        cp4.start()
        cp3.wait()
        cp4.wait()

        @plsc.parallel_loop(0, NVH, unroll=4)
        def _(i):
            sl = pl.ds(i * 16, 16)
            ka = kref[sl]
            kb = kpart[sl]
            va = vref[sl]
            vb = vpart[sl]
            take_a = (ka < kb) == want_min
            kref[sl] = jnp.where(take_a, ka, kb)
            vref[sl] = jnp.where(take_a, va, vb)

        _merge_finish(kref, vref)

    pltpu.sync_copy(vref, shv3.at[s])
    plsc.subcore_barrier()
    pltpu.sync_copy(shv3.at[partner], vpart)

    own_flagged = jnp.full((16,), h == 0)

    @plsc.parallel_loop(0, NVH, unroll=4)
    def _(i):
        sl = pl.ds(i * 16, 16)
        flags = ((_iota16_i32() + i * 16) < NUM_MASK).astype(jnp.int32)
        plsc.store_scatter(mref, [vref[sl]],
                           jnp.where(own_flagged, flags, 0))
        plsc.store_scatter(mref, [vpart[sl]],
                           jnp.where(own_flagged, 0, flags))

    @pl.when(h == 0)
    def _():
        @plsc.parallel_loop(0, NV, unroll=4, carry=jnp.int32(0))
        def _(i, cm):
            m = mref[pl.ds(i * 16, 16)]
            p = _iota16_i32() + i * 16
            cs = plsc.cumsum(m)
            plsc.store_scatter(mpref, [cm + cs - m], p, mask=m == 1)
            return cm + jnp.sum(m)

        pltpu.sync_copy(mref, mask_hbm.at[row])
        pltpu.sync_copy(mpref, mp_hbm.at[row])

    @pl.when(h == 1)
    def _():
        @plsc.parallel_loop(0, NV, unroll=4, carry=jnp.int32(0))
        def _(i, cv):
            m = mref[pl.ds(i * 16, 16)]
            p = _iota16_i32() + i * 16
            cs = plsc.cumsum(m)
            plsc.store_scatter(vpref, [cv + _iota16_i32() - (cs - m)], p,
                               mask=m == 0)
            return cv + (16 - jnp.sum(m))

        pltpu.sync_copy(vpref, vp_hbm.at[row])


@functools.cache
def _build():
    return pl.kernel(
        _body,
        out_type=(
            jax.ShapeDtypeStruct((B, S), jnp.int32),
            jax.ShapeDtypeStruct((B, MP_PAD), jnp.int32),
            jax.ShapeDtypeStruct((B, VP_PAD), jnp.int32),
        ),
        mesh=plsc.VectorSubcoreMesh(core_axis_name="c", subcore_axis_name="s",
                                    num_cores=2, num_subcores=16),
        compiler_params=pltpu.CompilerParams(needs_layout_passes=False),
        scratch_types=[
            pltpu.VMEM((H,), jnp.uint32),
            pltpu.VMEM((H,), jnp.uint32),
            pltpu.VMEM((H,), jnp.int32),
            pltpu.VMEM((H,), jnp.uint32),
            pltpu.VMEM((H,), jnp.int32),
            pltpu.VMEM((S,), jnp.int32),
            pltpu.VMEM((MP_PAD,), jnp.int32),
            pltpu.VMEM((VP_PAD,), jnp.int32),
            pltpu.VMEM_SHARED((16, H), jnp.uint32),
            pltpu.VMEM_SHARED((16, H), jnp.int32),
            pltpu.VMEM_SHARED((16, H), jnp.uint32),
            pltpu.VMEM_SHARED((16, H), jnp.int32),
            pltpu.VMEM_SHARED((16, H), jnp.int32),
            pltpu.SemaphoreType.DMA,
            pltpu.SemaphoreType.DMA,
        ],
    )


def kernel(x):
    assert x.shape == (B, S, 32)
    mask_i32, mp, vp = _build()()
    return (mask_i32.astype(bool), mp[:, :NUM_MASK], vp[:, :S - NUM_MASK])


# speedup vs baseline: 1.0060x; 1.0060x over previous
import functools

import jax
import jax.numpy as jnp
from jax import lax
from jax.experimental import pallas as pl
from jax.experimental.pallas import tpu as pltpu
from jax.experimental.pallas import tpu_sc as plsc

B = 16
S = 2048
H = S // 2
NUM_MASK = 614
NV = S // 16
NVH = H // 16
MP_PAD = 640
VP_PAD = 1440


def _rotl(v, r):
    return (v << jnp.uint32(r)) | (v >> jnp.uint32(32 - r))


def _tf2x32(k1, k2, x0, x1):
    ks2 = k1 ^ k2 ^ jnp.uint32(0x1BD11BDA)
    x0 = x0 + k1
    x1 = x1 + k2
    rot_a = (13, 15, 26, 6)
    rot_b = (17, 29, 16, 24)
    inj = ((k2, ks2, 1), (ks2, k1, 2), (k1, k2, 3), (k2, ks2, 4), (ks2, k1, 5))
    for i, (ka, kb, c) in enumerate(inj):
        for r in rot_a if i % 2 == 0 else rot_b:
            x0 = x0 + x1
            x1 = _rotl(x1, r)
            x1 = x0 ^ x1
        x0 = x0 + ka
        x1 = x1 + kb + jnp.uint32(c)
    return x0, x1


def _iota16_i32():
    return lax.iota(jnp.int32, 16)


def _tail_pass(kref, vref, k, flip):
    @plsc.parallel_loop(0, NVH, unroll=4)
    def _(b):
        sl = pl.ds(b * 16, 16)
        desc = (((b * 16) & k) != 0) != flip
        dm = jnp.full((16,), jnp.where(desc, jnp.uint32(0xFFFFFFFF),
                                       jnp.uint32(0)))
        ka, va = plsc.sort_key_val(kref[sl] ^ dm, vref[sl])
        kref[sl] = ka ^ dm
        vref[sl] = va


def _pair_pass(kref, vref, k, lj, flip):
    jv = 1 << lj

    @plsc.parallel_loop(0, NVH // 2, unroll=4)
    def _(m):
        a = ((m >> lj) << (lj + 1)) | (m & (jv - 1))
        b2 = a + jv
        asc = (((a * 16) & k) == 0) != flip
        sa = pl.ds(a * 16, 16)
        sb = pl.ds(b2 * 16, 16)
        ka = kref[sa]
        kb = kref[sb]
        va = vref[sa]
        vb = vref[sb]
        take_a = (ka < kb) == jnp.full((16,), asc)
        kref[sa] = jnp.where(take_a, ka, kb)
        kref[sb] = jnp.where(take_a, kb, ka)
        vref[sa] = jnp.where(take_a, va, vb)
        vref[sb] = jnp.where(take_a, vb, va)


def _sort_half(kref, vref, flip):
    _tail_pass(kref, vref, 16, flip)
    for k in (32, 64, 128, 256, 512, 1024):
        lj0 = (k // 32).bit_length() - 1

        def sub_body(t, _, k=k, lj0=lj0):
            _pair_pass(kref, vref, k, lj0 - t, flip)
            return 0

        lax.fori_loop(0, lj0 + 1, sub_body, 0, unroll=False)
        _tail_pass(kref, vref, k, flip)


def _merge_finish(kref, vref):
    def sub_body(t, _):
        _pair_pass(kref, vref, 2048, 5 - t, False)
        return 0

    lax.fori_loop(0, 6, sub_body, 0, unroll=False)
    _tail_pass(kref, vref, 2048, False)


def _body(mask_hbm, mp_hbm, vp_hbm,
          kref, k2ref, vref, kpart, vpart, mref, mpref, vpref,
          shk1, shv1, shk2, shv2, shv3, sem1, sem2):
    c = lax.axis_index("c")
    s = lax.axis_index("s")
    h = s // 8
    row = c * 8 + (s & 7)
    base = h * H
    flip = h == 1

    zero = jnp.zeros((16,), jnp.uint32)
    one = jnp.full((16,), 1, jnp.uint32)

    rk1, rk2 = _tf2x32(
        zero, jnp.full((16,), 42, jnp.uint32), zero,
        jnp.full((16,), row).astype(jnp.uint32))
    s1a, s1b = _tf2x32(rk1, rk2, zero, one)
    n1a, n1b = _tf2x32(rk1, rk2, zero, zero)
    s2a, s2b = _tf2x32(n1a, n1b, zero, one)

    @plsc.parallel_loop(0, NVH, unroll=1)
    def _(i):
        iota = _iota16_i32() + base + i * 16
        counts = iota.astype(jnp.uint32)
        b1, b2 = _tf2x32(s1a, s1b, zero, counts)
        c1, c2 = _tf2x32(s2a, s2b, zero, counts)
        sl = pl.ds(i * 16, 16)
        vref[sl] = iota
        kref[sl] = b1 ^ b2
        k2ref[sl] = c1 ^ c2

    want_min = jnp.full((16,), h == 0)
    partner = s ^ 8

    for rnd in range(2):
        shk, shv = (shk1, shv1) if rnd == 0 else (shk2, shv2)
        if rnd == 1:
            @plsc.parallel_loop(0, NVH, unroll=4)
            def _(i):
                sl = pl.ds(i * 16, 16)
                kref[sl] = k2ref[sl]

        _sort_half(kref, vref, flip)

        cp1 = pltpu.make_async_copy(kref, shk.at[s], sem1)
        cp1.start()
        cp2 = pltpu.make_async_copy(vref, shv.at[s], sem2)
        cp2.start()
        cp1.wait()
        cp2.wait()
        plsc.subcore_barrier()
        cp3 = pltpu.make_async_copy(shk.at[partner], kpart, sem1)
        cp3.start()
        cp4 = pltpu.make_async_copy(shv.at[partner], vpart, sem2)
        cp4.start()
        cp3.wait()
        cp4.wait()

        @plsc.parallel_loop(0, NVH, unroll=4)
        def _(i):
            sl = pl.ds(i * 16, 16)
            ka = kref[sl]
            kb = kpart[sl]
            va = vref[sl]
            vb = vpart[sl]
            take_a = (ka < kb) == want_min
            kref[sl] = jnp.where(take_a, ka, kb)
            vref[sl] = jnp.where(take_a, va, vb)

        _merge_finish(kref, vref)

    pltpu.sync_copy(vref, shv3.at[s])
    plsc.subcore_barrier()
    pltpu.sync_copy(shv3.at[partner], vpart)

    own_flagged = jnp.full((16,), h == 0)

    @plsc.parallel_loop(0, NVH, unroll=4)
    def _(i):
        sl = pl.ds(i * 16, 16)
        flags = ((_iota16_i32() + i * 16) < NUM_MASK).astype(jnp.int32)
        plsc.store_scatter(mref, [vref[sl]],
                           jnp.where(own_flagged, flags, 0))
        plsc.store_scatter(mref, [vpart[sl]],
                           jnp.where(own_flagged, 0, flags))

    @pl.when(h == 0)
    def _():
        @plsc.parallel_loop(0, NV, unroll=2, carry=jnp.int32(0))
        def _(i, cm):
            m = mref[pl.ds(i * 16, 16)]
            p = _iota16_i32() + i * 16
            cs = plsc.cumsum(m)
            plsc.store_scatter(mpref, [cm + cs - m], p, mask=m == 1)
            return cm + jnp.sum(m)

        pltpu.sync_copy(mref, mask_hbm.at[row])
        pltpu.sync_copy(mpref, mp_hbm.at[row])

    @pl.when(h == 1)
    def _():
        @plsc.parallel_loop(0, NV, unroll=2, carry=jnp.int32(0))
        def _(i, cv):
            m = mref[pl.ds(i * 16, 16)]
            p = _iota16_i32() + i * 16
            cs = plsc.cumsum(m)
            plsc.store_scatter(vpref, [cv + _iota16_i32() - (cs - m)], p,
                               mask=m == 0)
            return cv + (16 - jnp.sum(m))

        pltpu.sync_copy(vpref, vp_hbm.at[row])


@functools.cache
def _build():
    return pl.kernel(
        _body,
        out_type=(
            jax.ShapeDtypeStruct((B, S), jnp.int32),
            jax.ShapeDtypeStruct((B, MP_PAD), jnp.int32),
            jax.ShapeDtypeStruct((B, VP_PAD), jnp.int32),
        ),
        mesh=plsc.VectorSubcoreMesh(core_axis_name="c", subcore_axis_name="s",
                                    num_cores=2, num_subcores=16),
        compiler_params=pltpu.CompilerParams(needs_layout_passes=False),
        scratch_types=[
            pltpu.VMEM((H,), jnp.uint32),
            pltpu.VMEM((H,), jnp.uint32),
            pltpu.VMEM((H,), jnp.int32),
            pltpu.VMEM((H,), jnp.uint32),
            pltpu.VMEM((H,), jnp.int32),
            pltpu.VMEM((S,), jnp.int32),
            pltpu.VMEM((MP_PAD,), jnp.int32),
            pltpu.VMEM((VP_PAD,), jnp.int32),
            pltpu.VMEM_SHARED((16, H), jnp.uint32),
            pltpu.VMEM_SHARED((16, H), jnp.int32),
            pltpu.VMEM_SHARED((16, H), jnp.uint32),
            pltpu.VMEM_SHARED((16, H), jnp.int32),
            pltpu.VMEM_SHARED((16, H), jnp.int32),
            pltpu.SemaphoreType.DMA,
            pltpu.SemaphoreType.DMA,
        ],
    )


def kernel(x):
    assert x.shape == (B, S, 32)
    mask_i32, mp, vp = _build()()
    return (mask_i32.astype(bool), mp[:, :NUM_MASK], vp[:, :S - NUM_MASK])


# speedup vs baseline: 1.0359x; 1.0298x over previous
import functools

import jax
import jax.numpy as jnp
from jax import lax
from jax.experimental import pallas as pl
from jax.experimental.pallas import tpu as pltpu
from jax.experimental.pallas import tpu_sc as plsc

B = 16
S = 2048
H = S // 2
NUM_MASK = 614
NV = S // 16
NVH = H // 16
MP_PAD = 640
VP_PAD = 1440


def _rotl(v, r):
    return (v << jnp.uint32(r)) | (v >> jnp.uint32(32 - r))


def _tf2x32(k1, k2, x0, x1):
    ks2 = k1 ^ k2 ^ jnp.uint32(0x1BD11BDA)
    x0 = x0 + k1
    x1 = x1 + k2
    rot_a = (13, 15, 26, 6)
    rot_b = (17, 29, 16, 24)
    inj = ((k2, ks2, 1), (ks2, k1, 2), (k1, k2, 3), (k2, ks2, 4), (ks2, k1, 5))
    for i, (ka, kb, c) in enumerate(inj):
        for r in rot_a if i % 2 == 0 else rot_b:
            x0 = x0 + x1
            x1 = _rotl(x1, r)
            x1 = x0 ^ x1
        x0 = x0 + ka
        x1 = x1 + kb + jnp.uint32(c)
    return x0, x1


def _iota16_i32():
    return lax.iota(jnp.int32, 16)


def _tail_pass(kref, vref, k, flip):
    @plsc.parallel_loop(0, NVH, unroll=4)
    def _(b):
        sl = pl.ds(b * 16, 16)
        desc = (((b * 16) & k) != 0) != flip
        dm = jnp.full((16,), jnp.where(desc, jnp.uint32(0xFFFFFFFF),
                                       jnp.uint32(0)))
        ka, va = plsc.sort_key_val(kref[sl] ^ dm, vref[sl])
        kref[sl] = ka ^ dm
        vref[sl] = va


def _pair_pass(kref, vref, k, lj, flip):
    jv = 1 << lj

    @plsc.parallel_loop(0, NVH // 2, unroll=4)
    def _(m):
        a = ((m >> lj) << (lj + 1)) | (m & (jv - 1))
        b2 = a + jv
        asc = (((a * 16) & k) == 0) != flip
        sa = pl.ds(a * 16, 16)
        sb = pl.ds(b2 * 16, 16)
        ka = kref[sa]
        kb = kref[sb]
        va = vref[sa]
        vb = vref[sb]
        take_a = (ka < kb) == jnp.full((16,), asc)
        kref[sa] = jnp.where(take_a, ka, kb)
        kref[sb] = jnp.where(take_a, kb, ka)
        vref[sa] = jnp.where(take_a, va, vb)
        vref[sb] = jnp.where(take_a, vb, va)


def _sort_half(kref, vref, flip):
    _tail_pass(kref, vref, 16, flip)
    for k in (32, 64, 128, 256, 512, 1024):
        lj0 = (k // 32).bit_length() - 1

        def sub_body(t, _, k=k, lj0=lj0):
            _pair_pass(kref, vref, k, lj0 - t, flip)
            return 0

        lax.fori_loop(0, lj0 + 1, sub_body, 0, unroll=False)
        _tail_pass(kref, vref, k, flip)


def _tail_pass_keys(kref, k):
    @plsc.parallel_loop(0, NVH, unroll=4)
    def _(b):
        sl = pl.ds(b * 16, 16)
        desc = ((b * 16) & k) != 0
        dm = jnp.full((16,), jnp.where(desc, jnp.uint32(0xFFFFFFFF),
                                       jnp.uint32(0)))
        kref[sl] = jnp.sort(kref[sl] ^ dm) ^ dm


def _pair_pass_keys(kref, k, lj):
    jv = 1 << lj

    @plsc.parallel_loop(0, NVH // 2, unroll=4)
    def _(m):
        a = ((m >> lj) << (lj + 1)) | (m & (jv - 1))
        b2 = a + jv
        asc = ((a * 16) & k) == 0
        sa = pl.ds(a * 16, 16)
        sb = pl.ds(b2 * 16, 16)
        ka = kref[sa]
        kb = kref[sb]
        take_a = (ka < kb) == jnp.full((16,), asc)
        kref[sa] = jnp.where(take_a, ka, kb)
        kref[sb] = jnp.where(take_a, kb, ka)


def _sort_half_keys(kref):
    _tail_pass_keys(kref, 16)
    for k in (32, 64, 128, 256, 512, 1024):
        lj0 = (k // 32).bit_length() - 1

        def sub_body(t, _, k=k, lj0=lj0):
            _pair_pass_keys(kref, k, lj0 - t)
            return 0

        lax.fori_loop(0, lj0 + 1, sub_body, 0, unroll=False)
        _tail_pass_keys(kref, k)


def _merge_finish(kref, vref):
    def sub_body(t, _):
        _pair_pass(kref, vref, 2048, 5 - t, False)
        return 0

    lax.fori_loop(0, 6, sub_body, 0, unroll=False)
    _tail_pass(kref, vref, 2048, False)


def _body(mask_hbm, mp_hbm, vp_hbm,
          kref, k2ref, vref, kpart, vpart, mref, mpref, vpref,
          shk1, shv1, shk2, shv3, sem1, sem2):
    c = lax.axis_index("c")
    s = lax.axis_index("s")
    h = s // 8
    row = c * 8 + (s & 7)
    base = h * H
    flip = h == 1

    zero = jnp.zeros((16,), jnp.uint32)
    one = jnp.full((16,), 1, jnp.uint32)

    rk1, rk2 = _tf2x32(
        zero, jnp.full((16,), 42, jnp.uint32), zero,
        jnp.full((16,), row).astype(jnp.uint32))
    s1a, s1b = _tf2x32(rk1, rk2, zero, one)
    n1a, n1b = _tf2x32(rk1, rk2, zero, zero)
    s2a, s2b = _tf2x32(n1a, n1b, zero, one)

    @plsc.parallel_loop(0, NVH, unroll=1)
    def _(i):
        iota = _iota16_i32() + base + i * 16
        counts = iota.astype(jnp.uint32)
        b1, b2 = _tf2x32(s1a, s1b, zero, counts)
        c1, c2 = _tf2x32(s2a, s2b, zero, counts)
        sl = pl.ds(i * 16, 16)
        vref[sl] = iota
        kref[sl] = b1 ^ b2
        k2ref[sl] = c1 ^ c2

    want_min = jnp.full((16,), h == 0)
    partner = s ^ 8

    _sort_half(kref, vref, flip)

    cp1 = pltpu.make_async_copy(kref, shk1.at[s], sem1)
    cp1.start()
    cp2 = pltpu.make_async_copy(vref, shv1.at[s], sem2)
    cp2.start()
    cp1.wait()
    cp2.wait()
    plsc.subcore_barrier()
    cp3 = pltpu.make_async_copy(shk1.at[partner], kpart, sem1)
    cp3.start()
    cp4 = pltpu.make_async_copy(shv1.at[partner], vpart, sem2)
    cp4.start()
    cp3.wait()
    cp4.wait()

    @plsc.parallel_loop(0, NVH, unroll=4)
    def _(i):
        sl = pl.ds(i * 16, 16)
        ka = kref[sl]
        kb = kpart[sl]
        va = vref[sl]
        vb = vpart[sl]
        take_a = (ka < kb) == want_min
        kref[sl] = jnp.where(take_a, ka, kb)
        vref[sl] = jnp.where(take_a, va, vb)

    _merge_finish(kref, vref)

    @plsc.parallel_loop(0, NVH, unroll=4)
    def _(i):
        sl = pl.ds(i * 16, 16)
        kref[sl] = k2ref[sl]

    _sort_half_keys(kref)

    pltpu.sync_copy(kref, shk2.at[s])
    plsc.subcore_barrier()
    pltpu.sync_copy(shk2.at[partner], kpart)

    def _ld(ref, idx):
        vr = ref[pl.ds((idx >> 4) << 4, 16)]
        return jnp.sum(jnp.where(_iota16_i32() == (idx & 15), vr,
                                 jnp.uint32(0)))

    def bs_body(_, carry):
        lo, hi = carry
        mid = (lo + hi) // 2
        a_val = _ld(kref, jnp.minimum(mid, H - 1))
        b_val = _ld(kpart, jnp.clip(NUM_MASK - 1 - mid, 0, H - 1))
        g = jnp.where(NUM_MASK - 1 - mid >= 0, a_val < b_val,
                      jnp.bool_(False))
        return jnp.where(g, mid + 1, lo), jnp.where(g, hi, mid)

    istar, _ = lax.fori_loop(0, 11, bs_body,
                             (jnp.int32(0), jnp.int32(NUM_MASK)))
    a_top = _ld(kref, jnp.maximum(istar - 1, 0))
    b_top = _ld(kpart, jnp.clip(NUM_MASK - istar - 1, 0, H - 1))
    thr = jnp.maximum(jnp.where(istar > 0, a_top, jnp.uint32(0)),
                      jnp.where(istar < NUM_MASK, b_top, jnp.uint32(0)))
    thr_vec = jnp.full((16,), thr)

    @plsc.parallel_loop(0, NVH, unroll=2)
    def _(i):
        sl = pl.ds(i * 16, 16)
        flag = (k2ref[sl] <= thr_vec).astype(jnp.uint32)
        kref[sl] = vref[sl].astype(jnp.uint32) | (flag << jnp.uint32(12))

    pltpu.sync_copy(kref, shv3.at[s])
    plsc.subcore_barrier()
    pltpu.sync_copy(shv3.at[partner], kpart)

    @plsc.parallel_loop(0, NVH, unroll=4)
    def _(i):
        sl = pl.ds(i * 16, 16)
        fv_own = kref[sl]
        fv_par = kpart[sl]
        plsc.store_scatter(mref, [(fv_own & jnp.uint32(2047)).astype(jnp.int32)],
                           (fv_own >> jnp.uint32(12)).astype(jnp.int32))
        plsc.store_scatter(mref, [(fv_par & jnp.uint32(2047)).astype(jnp.int32)],
                           (fv_par >> jnp.uint32(12)).astype(jnp.int32))

    @pl.when(h == 0)
    def _():
        @plsc.parallel_loop(0, NV, unroll=2, carry=jnp.int32(0))
        def _(i, cm):
            m = mref[pl.ds(i * 16, 16)]
            p = _iota16_i32() + i * 16
            cs = plsc.cumsum(m)
            plsc.store_scatter(mpref, [cm + cs - m], p, mask=m == 1)
            return cm + jnp.sum(m)

        pltpu.sync_copy(mref, mask_hbm.at[row])
        pltpu.sync_copy(mpref, mp_hbm.at[row])

    @pl.when(h == 1)
    def _():
        @plsc.parallel_loop(0, NV, unroll=2, carry=jnp.int32(0))
        def _(i, cv):
            m = mref[pl.ds(i * 16, 16)]
            p = _iota16_i32() + i * 16
            cs = plsc.cumsum(m)
            plsc.store_scatter(vpref, [cv + _iota16_i32() - (cs - m)], p,
                               mask=m == 0)
            return cv + (16 - jnp.sum(m))

        pltpu.sync_copy(vpref, vp_hbm.at[row])


@functools.cache
def _build():
    return pl.kernel(
        _body,
        out_type=(
            jax.ShapeDtypeStruct((B, S), jnp.int32),
            jax.ShapeDtypeStruct((B, MP_PAD), jnp.int32),
            jax.ShapeDtypeStruct((B, VP_PAD), jnp.int32),
        ),
        mesh=plsc.VectorSubcoreMesh(core_axis_name="c", subcore_axis_name="s",
                                    num_cores=2, num_subcores=16),
        compiler_params=pltpu.CompilerParams(needs_layout_passes=False),
        scratch_types=[
            pltpu.VMEM((H,), jnp.uint32),
            pltpu.VMEM((H,), jnp.uint32),
            pltpu.VMEM((H,), jnp.int32),
            pltpu.VMEM((H,), jnp.uint32),
            pltpu.VMEM((H,), jnp.int32),
            pltpu.VMEM((S,), jnp.int32),
            pltpu.VMEM((MP_PAD,), jnp.int32),
            pltpu.VMEM((VP_PAD,), jnp.int32),
            pltpu.VMEM_SHARED((16, H), jnp.uint32),
            pltpu.VMEM_SHARED((16, H), jnp.int32),
            pltpu.VMEM_SHARED((16, H), jnp.uint32),
            pltpu.VMEM_SHARED((16, H), jnp.uint32),
            pltpu.SemaphoreType.DMA,
            pltpu.SemaphoreType.DMA,
        ],
    )


def kernel(x):
    assert x.shape == (B, S, 32)
    mask_i32, mp, vp = _build()()
    return (mask_i32.astype(bool), mp[:, :NUM_MASK], vp[:, :S - NUM_MASK])
